# baseline (speedup 1.0000x reference)
import jax
import jax.numpy as jnp
from jax import lax
from jax.experimental import pallas as pl
from jax.experimental.pallas import tpu as pltpu
from jax.experimental.pallas import tpu_sc as plsc

RSF = 2.5
NG = 4
TG = 2
TOPK = 2

NC, NS, L = 2, 16, 16
NW = NC * NS
NCH = 4
CHT = 512
CH = 64
BLK = 256
NB = 32
MAXR = NB * BLK
RPW = MAXR // NW
DUMP = 128


def _rank_lt_rows(vals, k):
    R = vals.shape[0]
    rows = []
    for j in range(R):
        col = vals[j : j + 1, :]
        gt = (vals > col).astype(jnp.float32)
        eq = (vals == col).astype(jnp.float32)
        eq_lo = sum([eq[i : i + 1, :] for i in range(j)]) if j else 0.0
        rows.append(jnp.sum(gt, axis=0, keepdims=True) + eq_lo)
    rank = jnp.concatenate(rows, axis=0)
    return (rank < float(k)).astype(jnp.float32)


def _router_kernel(x_ref, gw_ref, bias_ref, w13_ref, w2_ref,
                   wmat_ref, pos_ref, binfo_ref, bact_ref, xb_ref, sh_ref):
    e = pl.program_id(0)

    @pl.when(e == 0)
    def _routing():
        x = x_ref[...]
        xb_ref[...] = x.astype(jnp.bfloat16)
        E = gw_ref.shape[0]
        Tn = x.shape[0]
        per = E // NG
        f32 = jnp.float32
        i32 = jnp.int32
        logits = jax.lax.dot_general(
            gw_ref[...], x, (((1,), (1,)), ((), ())),
            preferred_element_type=f32)
        scores = jax.nn.sigmoid(logits)
        sfc = scores + bias_ref[...]
        gs = jnp.concatenate(
            [sum(sfc[per * g + i : per * g + i + 1, :] for i in range(per))
             for g in range(NG)], axis=0)
        gmask = _rank_lt_rows(gs, TG)
        emask = jnp.concatenate(
            [gmask[g : g + 1, :] for g in range(NG) for _ in range(per)],
            axis=0)
        masked = jnp.where(emask > 0.0, sfc, -1e30)
        chosen = _rank_lt_rows(masked, TOPK)
        w = scores * chosen
        w = w / (jnp.sum(w, axis=0, keepdims=True) + 1e-20)
        wmat_ref[...] = w * chosen * RSF

        r5 = lax.broadcasted_iota(i32, (CHT, CHT), 0)
        c5 = lax.broadcasted_iota(i32, (CHT, CHT), 1)
        tri = (r5 < c5).astype(f32)
        base_ec = jnp.zeros((E, 1), f32)
        tote = jnp.zeros((E, 1), f32)
        pref_chunks = []
        base_chunks = []
        for c in range(NCH):
            mc = chosen[:, c * CHT : (c + 1) * CHT]
            pc = jax.lax.dot_general(mc, tri, (((1,), (0,)), ((), ())),
                                     preferred_element_type=f32)
            pref_chunks.append(pc)
            base_chunks.append(base_ec)
            cnt = jnp.sum(mc, axis=1, keepdims=True)
            cnt64 = jnp.floor((cnt + (CH - 1)) * (1.0 / CH)) * CH
            base_ec = base_ec + cnt64
            tote = tote + cnt64
        rte = jnp.floor((tote + (BLK - 1)) * (1.0 / BLK)) * BLK
        segstart = jnp.zeros((E, 1), f32)
        acc = jnp.zeros((1, 1), f32)
        segs = []
        for j in range(E):
            segs.append(acc)
            acc = acc + rte[j : j + 1, :]
        segstart = jnp.concatenate(segs, axis=0)
        segend = segstart + rte
        pos_chunks = []
        for c in range(NCH):
            mc = chosen[:, c * CHT : (c + 1) * CHT]
            p = pref_chunks[c] + base_chunks[c] + segstart
            pos_chunks.append(jnp.where(mc > 0.0, p, -1.0))
        pos_ref[...] = jnp.concatenate(pos_chunks, axis=1).astype(i32)

        rowstart = (lax.broadcasted_iota(i32, (1, NW), 1) * BLK).astype(f32)
        bexp = jnp.zeros((1, NW), f32)
        for j in range(E):
            bexp = bexp + (rowstart >= segend[j : j + 1, :]).astype(f32)
        binfo_ref[...] = jnp.minimum(bexp, 7.0).astype(i32)
        bact_ref[...] = (bexp < 7.5).astype(i32)

    gu = jnp.dot(xb_ref[...], w13_ref[0], preferred_element_type=jnp.float32)
    dff = gu.shape[1] // 2
    g = gu[:, :dff]
    u = gu[:, dff:]
    h = (g * jax.nn.sigmoid(g)) * u
    contrib = jnp.dot(h.astype(jnp.bfloat16), w2_ref[0],
                      preferred_element_type=jnp.float32)

    @pl.when(e == 0)
    def _init():
        sh_ref[...] = contrib

    @pl.when(e != 0)
    def _acc():
        sh_ref[...] += contrib


def _scatter_body(pos_hbm, wmat_hbm, srctok_hbm, roww_hbm,
                  prow_v, wrow_v, idxs_v, vtok_v, vw_v, sem):
    i32 = jnp.int32
    c = lax.axis_index("c")
    s = lax.axis_index("s")
    wid = s * NC + c
    eid = wid // NCH
    chunk = wid - eid * NCH
    t0 = pl.multiple_of(chunk * CHT, CHT)
    lane = lax.broadcasted_iota(i32, (L,), 0)

    pltpu.sync_copy(pos_hbm.at[eid, pl.ds(t0, CHT)], prow_v)
    pltpu.sync_copy(wmat_hbm.at[eid, pl.ds(t0, CHT)], wrow_v)

    for i in range(CHT // L):
        p = prow_v[pl.ds(i * L, L)]
        m = p >= 0
        dump = MAXR + ((i * L) % DUMP) + lane
        r, o = divmod(i, 8)
        idxs_v[r, pl.ds(o * L, L)] = jnp.where(m, p, dump)
        vtok_v[r, pl.ds(o * L, L)] = t0 + i * L + lane
        vw_v[r, pl.ds(o * L, L)] = wrow_v[pl.ds(i * L, L)]
    for j in range(CHT // 128):
        pltpu.async_copy(vtok_v.at[j], srctok_hbm.at[idxs_v.at[j]],
                         sem).wait()
        pltpu.async_copy(vw_v.at[j], roww_hbm.at[idxs_v.at[j]], sem).wait()


def _gather_body(srctok_hbm, xb_hbm, xs_hbm, stbuf_v, idxg_v, rows_v, sem):
    i32 = jnp.int32
    c = lax.axis_index("c")
    s = lax.axis_index("s")
    wid = s * NC + c
    r0 = pl.multiple_of(wid * RPW, RPW)
    pltpu.sync_copy(srctok_hbm.at[pl.ds(r0, RPW)], stbuf_v)
    for k in range(RPW // CH):
        for j in range(CH // L):
            v = stbuf_v[pl.ds(k * CH + j * L, L)]
            idxg_v[pl.ds(j * L, L)] = jnp.minimum(jnp.maximum(v, 0), 2047)
        pltpu.async_copy(xb_hbm.at[idxg_v], rows_v, sem).wait()
        pltpu.sync_copy(rows_v,
                        xs_hbm.at[pl.ds(pl.multiple_of(r0 + k * CH, CH), CH)])


def _group_mm_kernel(binfo_ref, bact_ref, xs_ref, w13_ref, w2_ref, roww_ref,
                     ys_ref):
    b = pl.program_id(0)

    @pl.when(bact_ref[b] == 1)
    def _mm():
        gu = jnp.dot(xs_ref[...], w13_ref[0],
                     preferred_element_type=jnp.float32)
        dff = gu.shape[1] // 2
        g = gu[:, :dff]
        u = gu[:, dff:]
        h = (g * jax.nn.sigmoid(g)) * u
        y = jnp.dot(h.astype(jnp.bfloat16), w2_ref[0],
                    preferred_element_type=jnp.float32)
        ys_ref[...] = y * roww_ref[...]


def _combine_body(pos_hbm, ys_hbm, r1_hbm, r2_hbm,
                  posb_v, buf1_v, buf2_v, sem1, sem2):
    i32 = jnp.int32
    c = lax.axis_index("c")
    s = lax.axis_index("s")
    wid = s * NC + c
    tpw = 2048 // NW
    t0 = pl.multiple_of(wid * tpw, tpw)
    for e2 in range(8):
        pltpu.sync_copy(pos_hbm.at[e2, pl.ds(t0, tpw)], posb_v.at[e2])
    big = jnp.asarray(1 << 30, i32)
    for sub in range(tpw // L):
        p1 = jnp.zeros((L,), i32) + big
        p2 = jnp.zeros((L,), i32) - 1
        for e2 in range(8):
            pe = posb_v[e2, pl.ds(sub * L, L)]
            m = pe >= 0
            p1 = jnp.where(m, jnp.minimum(p1, pe), p1)
            p2 = jnp.where(m, jnp.maximum(p2, pe), p2)
        p1 = jnp.minimum(jnp.maximum(p1, 0), MAXR - 1)
        p2 = jnp.minimum(jnp.maximum(p2, 0), MAXR - 1)
        d1 = pltpu.async_copy(ys_hbm.at[p1], buf1_v, sem1)
        d2 = pltpu.async_copy(ys_hbm.at[p2], buf2_v, sem2)
        d1.wait()
        d2.wait()
        tt = pl.multiple_of(t0 + sub * L, L)
        pltpu.sync_copy(buf1_v, r1_hbm.at[pl.ds(tt, L)])
        pltpu.sync_copy(buf2_v, r2_hbm.at[pl.ds(tt, L)])


def _add_kernel(r1_ref, r2_ref, sh_ref, o_ref):
    o_ref[...] = r1_ref[...] + r2_ref[...] + sh_ref[...]


def kernel(hidden_states, gate_w, e_score_correction_bias, w13, w2,
           shared_w13, shared_w2):
    T, D = hidden_states.shape
    E, _, DFF2 = w13.shape
    DFF = DFF2 // 2
    SH = shared_w13.shape[1] // 2
    NSH = SH // DFF

    sg = shared_w13[:, :SH].reshape(D, NSH, DFF)
    su = shared_w13[:, SH:].reshape(D, NSH, DFF)
    sh13 = jnp.concatenate([sg, su], axis=-1).transpose(1, 0, 2)
    sh2 = shared_w2.reshape(NSH, DFF, D)
    bias2d = e_score_correction_bias.reshape(E, 1)

    f32 = jnp.float32
    i32 = jnp.int32
    wmat, pos, binfo, bact, xb, shared = pl.pallas_call(
        _router_kernel,
        grid=(NSH,),
        in_specs=[
            pl.BlockSpec((T, D), lambda e: (0, 0)),
            pl.BlockSpec((E, D), lambda e: (0, 0)),
            pl.BlockSpec((E, 1), lambda e: (0, 0)),
            pl.BlockSpec((1, D, DFF2), lambda e: (e, 0, 0)),
            pl.BlockSpec((1, DFF, D), lambda e: (e, 0, 0)),
        ],
        out_specs=[
            pl.BlockSpec((E, T), lambda e: (0, 0)),
            pl.BlockSpec((E, T), lambda e: (0, 0)),
            pl.BlockSpec((1, NW), lambda e: (0, 0)),
            pl.BlockSpec((1, NW), lambda e: (0, 0)),
            pl.BlockSpec((T, D), lambda e: (0, 0)),
            pl.BlockSpec((T, D), lambda e: (0, 0)),
        ],
        out_shape=[
            jax.ShapeDtypeStruct((E, T), f32),
            jax.ShapeDtypeStruct((E, T), i32),
            jax.ShapeDtypeStruct((1, NW), i32),
            jax.ShapeDtypeStruct((1, NW), i32),
            jax.ShapeDtypeStruct((T, D), jnp.bfloat16),
            jax.ShapeDtypeStruct((T, D), f32),
        ],
    )(hidden_states, gate_w, bias2d, sh13.astype(jnp.bfloat16),
      sh2.astype(jnp.bfloat16))

    mesh = plsc.VectorSubcoreMesh(core_axis_name="c", subcore_axis_name="s")
    xb_i32 = jax.lax.bitcast_convert_type(xb.reshape(T, 512, 2), i32)
    srctok, roww = pl.kernel(
        _scatter_body,
        out_type=[
            jax.ShapeDtypeStruct((MAXR + DUMP,), i32),
            jax.ShapeDtypeStruct((MAXR + DUMP,), f32),
        ],
        mesh=mesh,
        scratch_types=[
            pltpu.VMEM((CHT,), i32),
            pltpu.VMEM((CHT,), f32),
            pltpu.VMEM((CHT // 128, 128), i32),
            pltpu.VMEM((CHT // 128, 128), i32),
            pltpu.VMEM((CHT // 128, 128), f32),
            pltpu.SemaphoreType.DMA,
        ],
    )(pos, wmat)

    xs3 = pl.kernel(
        _gather_body,
        out_type=jax.ShapeDtypeStruct((MAXR, 512), i32),
        mesh=mesh,
        scratch_types=[
            pltpu.VMEM((RPW,), i32),
            pltpu.VMEM((CH,), i32),
            pltpu.VMEM((CH, 512), i32),
            pltpu.SemaphoreType.DMA,
        ],
    )(srctok, xb_i32)

    xs2d = jax.lax.bitcast_convert_type(
        xs3, jnp.bfloat16).reshape(MAXR, D)
    roww2d = roww[:MAXR].reshape(MAXR, 1)
    ys = pl.pallas_call(
        _group_mm_kernel,
        grid_spec=pltpu.PrefetchScalarGridSpec(
            num_scalar_prefetch=2,
            grid=(NB,),
            in_specs=[
                pl.BlockSpec((BLK, D), lambda b, bi, ba: (b, 0)),
                pl.BlockSpec((1, D, DFF2), lambda b, bi, ba: (bi[b], 0, 0)),
                pl.BlockSpec((1, DFF, D), lambda b, bi, ba: (bi[b], 0, 0)),
                pl.BlockSpec((BLK, 1), lambda b, bi, ba: (b, 0)),
            ],
            out_specs=pl.BlockSpec((BLK, D), lambda b, bi, ba: (b, 0)),
        ),
        out_shape=jax.ShapeDtypeStruct((MAXR, D), f32),
    )(binfo.reshape(NW), bact.reshape(NW), xs2d,
      w13.astype(jnp.bfloat16), w2.astype(jnp.bfloat16), roww2d)

    ys3 = ys.reshape(MAXR, 8, 128)
    r1, r2 = pl.kernel(
        _combine_body,
        out_type=[
            jax.ShapeDtypeStruct((T, 8, 128), f32),
            jax.ShapeDtypeStruct((T, 8, 128), f32),
        ],
        mesh=mesh,
        scratch_types=[
            pltpu.VMEM((E, T // NW), i32),
            pltpu.VMEM((L, 8, 128), f32),
            pltpu.VMEM((L, 8, 128), f32),
            pltpu.SemaphoreType.DMA,
            pltpu.SemaphoreType.DMA,
        ],
    )(pos, ys3)

    out = pl.pallas_call(
        _add_kernel,
        grid=(T // BLK,),
        in_specs=[
            pl.BlockSpec((BLK, D), lambda b: (b, 0)),
            pl.BlockSpec((BLK, D), lambda b: (b, 0)),
            pl.BlockSpec((BLK, D), lambda b: (b, 0)),
        ],
        out_specs=pl.BlockSpec((BLK, D), lambda b: (b, 0)),
        out_shape=jax.ShapeDtypeStruct((T, D), f32),
    )(r1.reshape(T, D), r2.reshape(T, D), shared)
    return out


# speedup vs baseline: 5.4775x; 5.4775x over previous
import jax
import jax.numpy as jnp
from jax import lax
from jax.experimental import pallas as pl
from jax.experimental.pallas import tpu as pltpu
from jax.experimental.pallas import tpu_sc as plsc

RSF = 2.5
NG = 4
TG = 2
TOPK = 2

NC, NS, L = 2, 16, 16
NW = NC * NS
NCH = 4
CHT = 512
CH = 64
BLK = 256
NB = 32
MAXR = NB * BLK
RPW = MAXR // NW
DUMP = 128


def _rank_lt_rows(vals, k):
    R = vals.shape[0]
    rows = []
    for j in range(R):
        col = vals[j : j + 1, :]
        gt = (vals > col).astype(jnp.float32)
        eq = (vals == col).astype(jnp.float32)
        eq_lo = sum([eq[i : i + 1, :] for i in range(j)]) if j else 0.0
        rows.append(jnp.sum(gt, axis=0, keepdims=True) + eq_lo)
    rank = jnp.concatenate(rows, axis=0)
    return (rank < float(k)).astype(jnp.float32)


def _router_kernel(x_ref, gw_ref, bias_ref, w13_ref, w2_ref,
                   w12_ref, pos_ref, binfo_ref, bact_ref, xb_ref, sh_ref):
    e = pl.program_id(0)

    @pl.when(e == 0)
    def _routing():
        x = x_ref[...]
        xb_ref[...] = x.astype(jnp.bfloat16)
        E = gw_ref.shape[0]
        Tn = x.shape[0]
        per = E // NG
        f32 = jnp.float32
        i32 = jnp.int32
        logits = jax.lax.dot_general(
            gw_ref[...], x, (((1,), (1,)), ((), ())),
            preferred_element_type=f32)
        scores = jax.nn.sigmoid(logits)
        sfc = scores + bias_ref[...]
        gs = jnp.concatenate(
            [sum(sfc[per * g + i : per * g + i + 1, :] for i in range(per))
             for g in range(NG)], axis=0)
        gmask = _rank_lt_rows(gs, TG)
        emask = jnp.concatenate(
            [gmask[g : g + 1, :] for g in range(NG) for _ in range(per)],
            axis=0)
        masked = jnp.where(emask > 0.0, sfc, -1e30)
        chosen = _rank_lt_rows(masked, TOPK)
        w = scores * chosen
        w = w / (jnp.sum(w, axis=0, keepdims=True) + 1e-20)
        wmat = w * chosen * RSF
        eidx = lax.broadcasted_iota(i32, (E, Tn), 0).astype(f32)
        emin = jnp.min(jnp.where(chosen > 0, eidx, 99.0), axis=0,
                       keepdims=True)
        emax = jnp.max(jnp.where(chosen > 0, eidx, -1.0), axis=0,
                       keepdims=True)
        w1 = jnp.sum(jnp.where(eidx == emin, wmat, 0.0), axis=0,
                     keepdims=True)
        w2_ = jnp.sum(jnp.where(eidx == emax, wmat, 0.0), axis=0,
                      keepdims=True)
        w12_ref[...] = jnp.concatenate([w1, w2_], axis=0)

        r5 = lax.broadcasted_iota(i32, (CHT, CHT), 0)
        c5 = lax.broadcasted_iota(i32, (CHT, CHT), 1)
        tri = (r5 < c5).astype(f32)
        base_ec = jnp.zeros((E, 1), f32)
        tote = jnp.zeros((E, 1), f32)
        pref_chunks = []
        base_chunks = []
        for c in range(NCH):
            mc = chosen[:, c * CHT : (c + 1) * CHT]
            pc = jax.lax.dot_general(mc, tri, (((1,), (0,)), ((), ())),
                                     preferred_element_type=f32)
            pref_chunks.append(pc)
            base_chunks.append(base_ec)
            cnt = jnp.sum(mc, axis=1, keepdims=True)
            cnt64 = jnp.floor((cnt + (CH - 1)) * (1.0 / CH)) * CH
            base_ec = base_ec + cnt64
            tote = tote + cnt64
        rte = jnp.floor((tote + (BLK - 1)) * (1.0 / BLK)) * BLK
        segstart = jnp.zeros((E, 1), f32)
        acc = jnp.zeros((1, 1), f32)
        segs = []
        for j in range(E):
            segs.append(acc)
            acc = acc + rte[j : j + 1, :]
        segstart = jnp.concatenate(segs, axis=0)
        segend = segstart + rte
        pos_chunks = []
        for c in range(NCH):
            mc = chosen[:, c * CHT : (c + 1) * CHT]
            p = pref_chunks[c] + base_chunks[c] + segstart
            pos_chunks.append(jnp.where(mc > 0.0, p, -1.0))
        pos_ref[...] = jnp.concatenate(pos_chunks, axis=1).astype(i32)

        rowstart = (lax.broadcasted_iota(i32, (1, NW), 1) * BLK).astype(f32)
        bexp = jnp.zeros((1, NW), f32)
        for j in range(E):
            bexp = bexp + (rowstart >= segend[j : j + 1, :]).astype(f32)
        binfo_ref[...] = jnp.minimum(bexp, 7.0).astype(i32)
        bact_ref[...] = (bexp < 7.5).astype(i32)

    gu = jnp.dot(xb_ref[...], w13_ref[0], preferred_element_type=jnp.float32)
    dff = gu.shape[1] // 2
    g = gu[:, :dff]
    u = gu[:, dff:]
    h = (g * jax.nn.sigmoid(g)) * u
    contrib = jnp.dot(h.astype(jnp.bfloat16), w2_ref[0],
                      preferred_element_type=jnp.float32)

    @pl.when(e == 0)
    def _init():
        sh_ref[...] = contrib

    @pl.when(e != 0)
    def _acc():
        sh_ref[...] += contrib


def _disperse_body(pos_hbm, xb_hbm, xs_hbm,
                   prow_v, idx_a, idx_b, rows_a, rows_b, sem_a, sem_b):
    i32 = jnp.int32
    c = lax.axis_index("c")
    s = lax.axis_index("s")
    wid = s * NC + c
    eid = wid // NCH
    chunk = wid - eid * NCH
    t0 = pl.multiple_of(chunk * CHT, CHT)
    lane = lax.broadcasted_iota(i32, (L,), 0)

    pltpu.sync_copy(pos_hbm.at[eid, pl.ds(t0, CHT)], prow_v)

    idxs = (idx_a, idx_b)
    bufs = (rows_a, rows_b)
    sems = (sem_a, sem_b)
    cps = [None, None]
    for k in range(CHT // CH):
        b = k % 2
        if cps[b] is not None:
            cps[b].wait()
        for j in range(CH // L):
            p = prow_v[pl.ds(k * CH + j * L, L)]
            dump = MAXR + ((k * CH + j * L) % DUMP) + lane
            idxs[b][pl.ds(j * L, L)] = jnp.where(p >= 0, p, dump)
        pltpu.sync_copy(
            xb_hbm.at[pl.ds(pl.multiple_of(t0 + k * CH, CH), CH)], bufs[b])
        cps[b] = pltpu.async_copy(bufs[b], xs_hbm.at[idxs[b]], sems[b])
    for cp in cps:
        if cp is not None:
            cp.wait()


def _group_mm_kernel(binfo_ref, bact_ref, xs_ref, w13_ref, w2_ref,
                     ys_ref):
    b = pl.program_id(0)

    @pl.when(bact_ref[b] == 1)
    def _mm():
        gu = jnp.dot(xs_ref[...], w13_ref[0],
                     preferred_element_type=jnp.float32)
        dff = gu.shape[1] // 2
        g = gu[:, :dff]
        u = gu[:, dff:]
        h = (g * jax.nn.sigmoid(g)) * u
        y = jnp.dot(h.astype(jnp.bfloat16), w2_ref[0],
                    preferred_element_type=jnp.float32)
        ys_ref[...] = y


def _combine_body(pos_hbm, ys_hbm, r1_hbm, r2_hbm,
                  posb_v, buf1_v, buf2_v, sem1, sem2):
    i32 = jnp.int32
    c = lax.axis_index("c")
    s = lax.axis_index("s")
    wid = s * NC + c
    tpw = 2048 // NW
    t0 = pl.multiple_of(wid * tpw, tpw)
    for e2 in range(8):
        pltpu.sync_copy(pos_hbm.at[e2, pl.ds(t0, tpw)], posb_v.at[e2])
    big = jnp.asarray(1 << 30, i32)
    for sub in range(tpw // L):
        p1 = jnp.zeros((L,), i32) + big
        p2 = jnp.zeros((L,), i32) - 1
        for e2 in range(8):
            pe = posb_v[e2, pl.ds(sub * L, L)]
            m = pe >= 0
            p1 = jnp.where(m, jnp.minimum(p1, pe), p1)
            p2 = jnp.where(m, jnp.maximum(p2, pe), p2)
        p1 = jnp.minimum(jnp.maximum(p1, 0), MAXR - 1)
        p2 = jnp.minimum(jnp.maximum(p2, 0), MAXR - 1)
        d1 = pltpu.async_copy(ys_hbm.at[p1], buf1_v, sem1)
        d2 = pltpu.async_copy(ys_hbm.at[p2], buf2_v, sem2)
        d1.wait()
        d2.wait()
        tt = pl.multiple_of(t0 + sub * L, L)
        pltpu.sync_copy(buf1_v, r1_hbm.at[pl.ds(tt, L)])
        pltpu.sync_copy(buf2_v, r2_hbm.at[pl.ds(tt, L)])


def _add_kernel(w12_ref, r1_ref, r2_ref, sh_ref, o_ref):
    o_ref[...] = (w12_ref[:, 0:1] * r1_ref[...]
                  + w12_ref[:, 1:2] * r2_ref[...] + sh_ref[...])


def kernel(hidden_states, gate_w, e_score_correction_bias, w13, w2,
           shared_w13, shared_w2):
    T, D = hidden_states.shape
    E, _, DFF2 = w13.shape
    DFF = DFF2 // 2
    SH = shared_w13.shape[1] // 2
    NSH = SH // DFF

    sg = shared_w13[:, :SH].reshape(D, NSH, DFF)
    su = shared_w13[:, SH:].reshape(D, NSH, DFF)
    sh13 = jnp.concatenate([sg, su], axis=-1).transpose(1, 0, 2)
    sh2 = shared_w2.reshape(NSH, DFF, D)
    bias2d = e_score_correction_bias.reshape(E, 1)

    f32 = jnp.float32
    i32 = jnp.int32
    w12, pos, binfo, bact, xb, shared = pl.pallas_call(
        _router_kernel,
        grid=(NSH,),
        in_specs=[
            pl.BlockSpec((T, D), lambda e: (0, 0)),
            pl.BlockSpec((E, D), lambda e: (0, 0)),
            pl.BlockSpec((E, 1), lambda e: (0, 0)),
            pl.BlockSpec((1, D, DFF2), lambda e: (e, 0, 0)),
            pl.BlockSpec((1, DFF, D), lambda e: (e, 0, 0)),
        ],
        out_specs=[
            pl.BlockSpec((2, T), lambda e: (0, 0)),
            pl.BlockSpec((E, T), lambda e: (0, 0)),
            pl.BlockSpec((1, NW), lambda e: (0, 0)),
            pl.BlockSpec((1, NW), lambda e: (0, 0)),
            pl.BlockSpec((T, D), lambda e: (0, 0)),
            pl.BlockSpec((T, D), lambda e: (0, 0)),
        ],
        out_shape=[
            jax.ShapeDtypeStruct((2, T), f32),
            jax.ShapeDtypeStruct((E, T), i32),
            jax.ShapeDtypeStruct((1, NW), i32),
            jax.ShapeDtypeStruct((1, NW), i32),
            jax.ShapeDtypeStruct((T, D), jnp.bfloat16),
            jax.ShapeDtypeStruct((T, D), f32),
        ],
    )(hidden_states, gate_w, bias2d, sh13.astype(jnp.bfloat16),
      sh2.astype(jnp.bfloat16))

    mesh = plsc.VectorSubcoreMesh(core_axis_name="c", subcore_axis_name="s")
    xb_i32 = jax.lax.bitcast_convert_type(xb.reshape(T, 512, 2), i32)
    xs3 = pl.kernel(
        _disperse_body,
        out_type=jax.ShapeDtypeStruct((MAXR + DUMP, 512), i32),
        mesh=mesh,
        scratch_types=[
            pltpu.VMEM((CHT,), i32),
            pltpu.VMEM((CH,), i32),
            pltpu.VMEM((CH,), i32),
            pltpu.VMEM((CH, 512), i32),
            pltpu.VMEM((CH, 512), i32),
            pltpu.SemaphoreType.DMA,
            pltpu.SemaphoreType.DMA,
        ],
    )(pos, xb_i32)

    xs2d = jax.lax.bitcast_convert_type(
        xs3[:MAXR], jnp.bfloat16).reshape(MAXR, D)
    ys = pl.pallas_call(
        _group_mm_kernel,
        grid_spec=pltpu.PrefetchScalarGridSpec(
            num_scalar_prefetch=2,
            grid=(NB,),
            in_specs=[
                pl.BlockSpec((BLK, D), lambda b, bi, ba: (b, 0)),
                pl.BlockSpec((1, D, DFF2), lambda b, bi, ba: (bi[b], 0, 0)),
                pl.BlockSpec((1, DFF, D), lambda b, bi, ba: (bi[b], 0, 0)),
            ],
            out_specs=pl.BlockSpec((BLK, D), lambda b, bi, ba: (b, 0)),
        ),
        out_shape=jax.ShapeDtypeStruct((MAXR, D), f32),
    )(binfo.reshape(NW), bact.reshape(NW), xs2d,
      w13.astype(jnp.bfloat16), w2.astype(jnp.bfloat16))

    ys3 = ys.reshape(MAXR, 8, 128)
    r1, r2 = pl.kernel(
        _combine_body,
        out_type=[
            jax.ShapeDtypeStruct((T, 8, 128), f32),
            jax.ShapeDtypeStruct((T, 8, 128), f32),
        ],
        mesh=mesh,
        scratch_types=[
            pltpu.VMEM((E, T // NW), i32),
            pltpu.VMEM((L, 8, 128), f32),
            pltpu.VMEM((L, 8, 128), f32),
            pltpu.SemaphoreType.DMA,
            pltpu.SemaphoreType.DMA,
        ],
    )(pos, ys3)

    out = pl.pallas_call(
        _add_kernel,
        grid=(T // BLK,),
        in_specs=[
            pl.BlockSpec((BLK, 2), lambda b: (b, 0)),
            pl.BlockSpec((BLK, D), lambda b: (b, 0)),
            pl.BlockSpec((BLK, D), lambda b: (b, 0)),
            pl.BlockSpec((BLK, D), lambda b: (b, 0)),
        ],
        out_specs=pl.BlockSpec((BLK, D), lambda b: (b, 0)),
        out_shape=jax.ShapeDtypeStruct((T, D), f32),
    )(w12.T, r1.reshape(T, D), r2.reshape(T, D), shared)
    return out


# speedup vs baseline: 5.5210x; 1.0079x over previous
import jax
import jax.numpy as jnp
from jax import lax
from jax.experimental import pallas as pl
from jax.experimental.pallas import tpu as pltpu
from jax.experimental.pallas import tpu_sc as plsc

RSF = 2.5
NG = 4
TG = 2
TOPK = 2

NC, NS, L = 2, 16, 16
NW = NC * NS
NCH = 4
CHT = 512
CH = 64
BLK = 256
NB = 32
MAXR = NB * BLK
RPW = MAXR // NW
DUMP = 128


def _rank_lt_rows(vals, k):
    R = vals.shape[0]
    rows = []
    for j in range(R):
        col = vals[j : j + 1, :]
        gt = (vals > col).astype(jnp.float32)
        eq = (vals == col).astype(jnp.float32)
        eq_lo = sum([eq[i : i + 1, :] for i in range(j)]) if j else 0.0
        rows.append(jnp.sum(gt, axis=0, keepdims=True) + eq_lo)
    rank = jnp.concatenate(rows, axis=0)
    return (rank < float(k)).astype(jnp.float32)


def _router_kernel(x_ref, gw_ref, bias_ref, w13_ref, w2_ref,
                   w12_ref, pos_ref, binfo_ref, bact_ref, xb_ref, sh_ref):
    e = pl.program_id(0)

    @pl.when(e == 0)
    def _routing():
        x = x_ref[...]
        xb_ref[...] = x.astype(jnp.bfloat16)
        E = gw_ref.shape[0]
        Tn = x.shape[0]
        per = E // NG
        f32 = jnp.float32
        i32 = jnp.int32
        logits = jax.lax.dot_general(
            gw_ref[...], x, (((1,), (1,)), ((), ())),
            preferred_element_type=f32)
        scores = jax.nn.sigmoid(logits)
        sfc = scores + bias_ref[...]
        gs = jnp.concatenate(
            [sum(sfc[per * g + i : per * g + i + 1, :] for i in range(per))
             for g in range(NG)], axis=0)
        gmask = _rank_lt_rows(gs, TG)
        emask = jnp.concatenate(
            [gmask[g : g + 1, :] for g in range(NG) for _ in range(per)],
            axis=0)
        masked = jnp.where(emask > 0.0, sfc, -1e30)
        chosen = _rank_lt_rows(masked, TOPK)
        w = scores * chosen
        w = w / (jnp.sum(w, axis=0, keepdims=True) + 1e-20)
        wmat = w * chosen * RSF
        eidx = lax.broadcasted_iota(i32, (E, Tn), 0).astype(f32)
        emin = jnp.min(jnp.where(chosen > 0, eidx, 99.0), axis=0,
                       keepdims=True)
        emax = jnp.max(jnp.where(chosen > 0, eidx, -1.0), axis=0,
                       keepdims=True)
        w1 = jnp.sum(jnp.where(eidx == emin, wmat, 0.0), axis=0,
                     keepdims=True)
        w2_ = jnp.sum(jnp.where(eidx == emax, wmat, 0.0), axis=0,
                      keepdims=True)
        w12_ref[...] = jnp.concatenate([w1, w2_], axis=0)

        r5 = lax.broadcasted_iota(i32, (CHT, CHT), 0)
        c5 = lax.broadcasted_iota(i32, (CHT, CHT), 1)
        tri = (r5 < c5).astype(f32)
        base_ec = jnp.zeros((E, 1), f32)
        tote = jnp.zeros((E, 1), f32)
        pref_chunks = []
        base_chunks = []
        for c in range(NCH):
            mc = chosen[:, c * CHT : (c + 1) * CHT]
            pc = jax.lax.dot_general(mc, tri, (((1,), (0,)), ((), ())),
                                     preferred_element_type=f32)
            pref_chunks.append(pc)
            base_chunks.append(base_ec)
            cnt = jnp.sum(mc, axis=1, keepdims=True)
            cnt64 = jnp.floor((cnt + (CH - 1)) * (1.0 / CH)) * CH
            base_ec = base_ec + cnt64
            tote = tote + cnt64
        rte = jnp.floor((tote + (BLK - 1)) * (1.0 / BLK)) * BLK
        segstart = jnp.zeros((E, 1), f32)
        acc = jnp.zeros((1, 1), f32)
        segs = []
        for j in range(E):
            segs.append(acc)
            acc = acc + rte[j : j + 1, :]
        segstart = jnp.concatenate(segs, axis=0)
        segend = segstart + rte
        pos_chunks = []
        for c in range(NCH):
            mc = chosen[:, c * CHT : (c + 1) * CHT]
            p = pref_chunks[c] + base_chunks[c] + segstart
            pos_chunks.append(jnp.where(mc > 0.0, p, -1.0))
        pos_ref[...] = jnp.concatenate(pos_chunks, axis=1).astype(i32)

        rowstart = (lax.broadcasted_iota(i32, (1, NW), 1) * BLK).astype(f32)
        bexp = jnp.zeros((1, NW), f32)
        for j in range(E):
            bexp = bexp + (rowstart >= segend[j : j + 1, :]).astype(f32)
        binfo_ref[...] = jnp.minimum(bexp, 7.0).astype(i32)
        bact_ref[...] = (bexp < 7.5).astype(i32)

    gu = jnp.dot(xb_ref[...], w13_ref[0], preferred_element_type=jnp.float32)
    dff = gu.shape[1] // 2
    g = gu[:, :dff]
    u = gu[:, dff:]
    h = (g * jax.nn.sigmoid(g)) * u
    contrib = jnp.dot(h.astype(jnp.bfloat16), w2_ref[0],
                      preferred_element_type=jnp.float32)

    @pl.when(e == 0)
    def _init():
        sh_ref[...] = contrib

    @pl.when(e != 0)
    def _acc():
        sh_ref[...] += contrib


def _disperse_body(pos_hbm, xb_hbm, xs_hbm,
                   prow_v, idx_a, idx_b, rows_a, rows_b, sem_a, sem_b):
    i32 = jnp.int32
    c = lax.axis_index("c")
    s = lax.axis_index("s")
    wid = s * NC + c
    eid = wid // NCH
    chunk = wid - eid * NCH
    t0 = pl.multiple_of(chunk * CHT, CHT)
    lane = lax.broadcasted_iota(i32, (L,), 0)

    pltpu.sync_copy(pos_hbm.at[eid, pl.ds(t0, CHT)], prow_v)

    idxs = (idx_a, idx_b)
    bufs = (rows_a, rows_b)
    sems = (sem_a, sem_b)
    cps = [None, None]
    for k in range(CHT // CH):
        b = k % 2
        if cps[b] is not None:
            cps[b].wait()
        for j in range(CH // L):
            p = prow_v[pl.ds(k * CH + j * L, L)]
            dump = MAXR + ((k * CH + j * L) % DUMP) + lane
            idxs[b][pl.ds(j * L, L)] = jnp.where(p >= 0, p, dump)
        pltpu.sync_copy(
            xb_hbm.at[pl.ds(pl.multiple_of(t0 + k * CH, CH), CH)], bufs[b])
        cps[b] = pltpu.async_copy(bufs[b], xs_hbm.at[idxs[b]], sems[b])
    for cp in cps:
        if cp is not None:
            cp.wait()


def _group_mm_kernel(binfo_ref, bact_ref, xs_ref, w13_ref, w2_ref,
                     ys_ref):
    b = pl.program_id(0)

    @pl.when(bact_ref[b] == 1)
    def _mm():
        gu = jnp.dot(xs_ref[...], w13_ref[0],
                     preferred_element_type=jnp.float32)
        dff = gu.shape[1] // 2
        g = gu[:, :dff]
        u = gu[:, dff:]
        h = (g * jax.nn.sigmoid(g)) * u
        y = jnp.dot(h.astype(jnp.bfloat16), w2_ref[0],
                    preferred_element_type=jnp.float32)
        ys_ref[...] = y


def _combine_body(pos_hbm, ys_hbm, r1_hbm, r2_hbm,
                  posb_v, buf1_v, buf2_v, sem1, sem2):
    i32 = jnp.int32
    c = lax.axis_index("c")
    s = lax.axis_index("s")
    wid = s * NC + c
    tpw = 2048 // NW
    t0 = pl.multiple_of(wid * tpw, tpw)
    for e2 in range(8):
        pltpu.sync_copy(pos_hbm.at[e2, pl.ds(t0, tpw)], posb_v.at[e2])
    big = jnp.asarray(1 << 30, i32)
    for sub in range(tpw // L):
        p1 = jnp.zeros((L,), i32) + big
        p2 = jnp.zeros((L,), i32) - 1
        for e2 in range(8):
            pe = posb_v[e2, pl.ds(sub * L, L)]
            m = pe >= 0
            p1 = jnp.where(m, jnp.minimum(p1, pe), p1)
            p2 = jnp.where(m, jnp.maximum(p2, pe), p2)
        p1 = jnp.minimum(jnp.maximum(p1, 0), MAXR - 1)
        p2 = jnp.minimum(jnp.maximum(p2, 0), MAXR - 1)
        d1 = pltpu.async_copy(ys_hbm.at[p1], buf1_v, sem1)
        d2 = pltpu.async_copy(ys_hbm.at[p2], buf2_v, sem2)
        d1.wait()
        d2.wait()
        tt = pl.multiple_of(t0 + sub * L, L)
        pltpu.sync_copy(buf1_v, r1_hbm.at[pl.ds(tt, L)])
        pltpu.sync_copy(buf2_v, r2_hbm.at[pl.ds(tt, L)])


def _add_kernel(w12_ref, r1_ref, r2_ref, sh_ref, o_ref):
    o_ref[...] = (w12_ref[:, 0:1] * r1_ref[...]
                  + w12_ref[:, 1:2] * r2_ref[...] + sh_ref[...])


def kernel(hidden_states, gate_w, e_score_correction_bias, w13, w2,
           shared_w13, shared_w2):
    T, D = hidden_states.shape
    E, _, DFF2 = w13.shape
    DFF = DFF2 // 2
    SH = shared_w13.shape[1] // 2
    NSH = SH // DFF

    sg = shared_w13[:, :SH].reshape(D, NSH, DFF)
    su = shared_w13[:, SH:].reshape(D, NSH, DFF)
    sh13 = jnp.concatenate([sg, su], axis=-1).transpose(1, 0, 2)
    sh2 = shared_w2.reshape(NSH, DFF, D)
    bias2d = e_score_correction_bias.reshape(E, 1)

    f32 = jnp.float32
    i32 = jnp.int32
    w12, pos, binfo, bact, xb, shared = pl.pallas_call(
        _router_kernel,
        grid=(NSH,),
        in_specs=[
            pl.BlockSpec((T, D), lambda e: (0, 0)),
            pl.BlockSpec((E, D), lambda e: (0, 0)),
            pl.BlockSpec((E, 1), lambda e: (0, 0)),
            pl.BlockSpec((1, D, DFF2), lambda e: (e, 0, 0)),
            pl.BlockSpec((1, DFF, D), lambda e: (e, 0, 0)),
        ],
        out_specs=[
            pl.BlockSpec((2, T), lambda e: (0, 0)),
            pl.BlockSpec((E, T), lambda e: (0, 0)),
            pl.BlockSpec((1, NW), lambda e: (0, 0)),
            pl.BlockSpec((1, NW), lambda e: (0, 0)),
            pl.BlockSpec((T, D), lambda e: (0, 0)),
            pl.BlockSpec((T, D), lambda e: (0, 0)),
        ],
        out_shape=[
            jax.ShapeDtypeStruct((2, T), f32),
            jax.ShapeDtypeStruct((E, T), i32),
            jax.ShapeDtypeStruct((1, NW), i32),
            jax.ShapeDtypeStruct((1, NW), i32),
            jax.ShapeDtypeStruct((T, D), jnp.bfloat16),
            jax.ShapeDtypeStruct((T, D), f32),
        ],
    )(hidden_states, gate_w, bias2d, sh13.astype(jnp.bfloat16),
      sh2.astype(jnp.bfloat16))

    mesh = plsc.VectorSubcoreMesh(core_axis_name="c", subcore_axis_name="s")
    xb_i32 = jax.lax.bitcast_convert_type(xb.reshape(T, 512, 2), i32)
    xs3 = pl.kernel(
        _disperse_body,
        out_type=jax.ShapeDtypeStruct((MAXR + DUMP, 512), i32),
        mesh=mesh,
        compiler_params=pltpu.CompilerParams(use_tc_tiling_on_sc=True),
        scratch_types=[
            pltpu.VMEM((CHT,), i32),
            pltpu.VMEM((CH,), i32),
            pltpu.VMEM((CH,), i32),
            pltpu.VMEM((CH, 512), i32),
            pltpu.VMEM((CH, 512), i32),
            pltpu.SemaphoreType.DMA,
            pltpu.SemaphoreType.DMA,
        ],
    )(pos, xb_i32)

    xs2d = jax.lax.bitcast_convert_type(
        xs3[:MAXR], jnp.bfloat16).reshape(MAXR, D)
    ys = pl.pallas_call(
        _group_mm_kernel,
        grid_spec=pltpu.PrefetchScalarGridSpec(
            num_scalar_prefetch=2,
            grid=(NB,),
            in_specs=[
                pl.BlockSpec((BLK, D), lambda b, bi, ba: (b, 0)),
                pl.BlockSpec((1, D, DFF2), lambda b, bi, ba: (bi[b], 0, 0)),
                pl.BlockSpec((1, DFF, D), lambda b, bi, ba: (bi[b], 0, 0)),
            ],
            out_specs=pl.BlockSpec((BLK, D), lambda b, bi, ba: (b, 0)),
        ),
        out_shape=jax.ShapeDtypeStruct((MAXR, D), f32),
    )(binfo.reshape(NW), bact.reshape(NW), xs2d,
      w13.astype(jnp.bfloat16), w2.astype(jnp.bfloat16))

    ys3 = ys.reshape(MAXR, 8, 128)
    r1, r2 = pl.kernel(
        _combine_body,
        out_type=[
            jax.ShapeDtypeStruct((T, 8, 128), f32),
            jax.ShapeDtypeStruct((T, 8, 128), f32),
        ],
        mesh=mesh,
        compiler_params=pltpu.CompilerParams(use_tc_tiling_on_sc=True),
        scratch_types=[
            pltpu.VMEM((E, T // NW), i32),
            pltpu.VMEM((L, 8, 128), f32),
            pltpu.VMEM((L, 8, 128), f32),
            pltpu.SemaphoreType.DMA,
            pltpu.SemaphoreType.DMA,
        ],
    )(pos, ys3)

    out = pl.pallas_call(
        _add_kernel,
        grid=(T // BLK,),
        in_specs=[
            pl.BlockSpec((BLK, 2), lambda b: (b, 0)),
            pl.BlockSpec((BLK, D), lambda b: (b, 0)),
            pl.BlockSpec((BLK, D), lambda b: (b, 0)),
            pl.BlockSpec((BLK, D), lambda b: (b, 0)),
        ],
        out_specs=pl.BlockSpec((BLK, D), lambda b: (b, 0)),
        out_shape=jax.ShapeDtypeStruct((T, D), f32),
    )(w12.T, r1.reshape(T, D), r2.reshape(T, D), shared)
    return out


# speedup vs baseline: 15.6174x; 2.8287x over previous
import numpy as np

import jax
import jax.numpy as jnp
from jax.experimental import pallas as pl
from jax.experimental.pallas import tpu as pltpu

RSF = 2.5
NG = 4
TG = 2
TOPK = 2


def _rank_lt(vals, k):
    Tn, L = vals.shape
    lane = jax.lax.broadcasted_iota(jnp.int32, (Tn, L), 1)
    cols = []
    for j in range(L):
        col = vals[:, j : j + 1]
        gt = (vals > col).astype(jnp.float32)
        eq_lo = jnp.logical_and(vals == col, lane < j).astype(jnp.float32)
        cols.append(jnp.sum(gt + eq_lo, axis=1, keepdims=True))
    rank = jnp.concatenate(cols, axis=1)
    return (rank < float(k)).astype(jnp.float32)


def _moe_kernel(x_ref, gw_ref, bias_ref, w13_ref, w2_ref, o_ref, cw_ref,
                xb_ref):
    e = pl.program_id(0)

    @pl.when(e == 0)
    def _routing():
        x = x_ref[...]
        xb_ref[...] = x.astype(jnp.bfloat16)
        E = gw_ref.shape[0]
        per = E // NG
        logits = jax.lax.dot_general(
            x, gw_ref[...], (((1,), (1,)), ((), ())),
            preferred_element_type=jnp.float32)
        scores = jax.nn.sigmoid(logits)
        sfc = scores + bias_ref[...]
        lane_e = jax.lax.broadcasted_iota(jnp.int32, (x.shape[0], E), 1)
        grp_of_e = lane_e // per
        gs = jnp.concatenate(
            [jnp.sum(jnp.where(grp_of_e == g, sfc, 0.0), axis=1,
                     keepdims=True) for g in range(NG)], axis=1)
        gmask = _rank_lt(gs, TG)
        emask = jnp.concatenate(
            [jnp.broadcast_to(gmask[:, g : g + 1], (x.shape[0], per))
             for g in range(NG)], axis=1)
        masked = jnp.where(emask > 0.0, sfc, -jnp.inf)
        chosen = _rank_lt(masked, TOPK)
        w = scores * chosen
        w = w / (jnp.sum(w, axis=1, keepdims=True) + 1e-20)
        cw_ref[:, :E] = w * RSF
        cw_ref[:, E:] = jnp.ones((x.shape[0], cw_ref.shape[1] - E),
                                 jnp.float32)

    gu = jnp.dot(xb_ref[...], w13_ref[0], preferred_element_type=jnp.float32)
    dff = gu.shape[1] // 2
    g = gu[:, :dff]
    u = gu[:, dff:]
    h = (g * jax.nn.sigmoid(g)) * u
    contrib = jnp.dot(h.astype(jnp.bfloat16), w2_ref[0],
                      preferred_element_type=jnp.float32)
    lane = jax.lax.broadcasted_iota(jnp.int32, cw_ref.shape, 1)
    wcol = jnp.sum(jnp.where(lane == e, cw_ref[...], 0.0), axis=1,
                   keepdims=True)
    contrib = contrib * wcol

    @pl.when(e == 0)
    def _init():
        o_ref[...] = contrib

    @pl.when(e != 0)
    def _acc():
        o_ref[...] += contrib


def kernel(hidden_states, gate_w, e_score_correction_bias, w13, w2,
           shared_w13, shared_w2):
    T, D = hidden_states.shape
    E, _, DFF2 = w13.shape
    DFF = DFF2 // 2
    SH = shared_w13.shape[1] // 2
    NSH = SH // DFF

    sg = shared_w13[:, :SH].reshape(D, NSH, DFF)
    su = shared_w13[:, SH:].reshape(D, NSH, DFF)
    sh13 = jnp.concatenate([sg, su], axis=-1).transpose(1, 0, 2)
    sh2 = shared_w2.reshape(NSH, DFF, D)
    w13_all = jnp.concatenate([w13, sh13], axis=0)
    w2_all = jnp.concatenate([w2, sh2], axis=0)
    NE = E + NSH

    bias2d = e_score_correction_bias.reshape(1, E)

    out = pl.pallas_call(
        _moe_kernel,
        grid=(NE,),
        in_specs=[
            pl.BlockSpec((T, D), lambda e: (0, 0)),
            pl.BlockSpec((E, D), lambda e: (0, 0)),
            pl.BlockSpec((1, E), lambda e: (0, 0)),
            pl.BlockSpec((1, D, DFF2), lambda e: (e, 0, 0)),
            pl.BlockSpec((1, DFF, D), lambda e: (e, 0, 0)),
        ],
        out_specs=pl.BlockSpec((T, D), lambda e: (0, 0)),
        out_shape=jax.ShapeDtypeStruct((T, D), hidden_states.dtype),
        scratch_shapes=[pltpu.VMEM((T, NE), jnp.float32),
                        pltpu.VMEM((T, D), jnp.bfloat16)],
    )(hidden_states, gate_w, bias2d, w13_all.astype(jnp.bfloat16),
      w2_all.astype(jnp.bfloat16))
    return out
